# baseline (speedup 1.0000x reference)
import functools

import jax
import jax.numpy as jnp
from jax import lax
from jax.experimental import pallas as pl
from jax.experimental.pallas import tpu as pltpu
from jax.experimental.pallas import tpu_sc as plsc

_B = 32
_F = 32 * 32 * 768
_CH = 32768
_UN = 16
_NC = 2


@functools.partial(
    pl.kernel,
    mesh=plsc.VectorSubcoreMesh(core_axis_name="c", subcore_axis_name="s"),
    out_type=jax.ShapeDtypeStruct((_B, _F), jnp.float32),
    scratch_types=[
        pltpu.VMEM((_B,), jnp.int32),
        pltpu.VMEM((_B, 128), jnp.float32),
        pltpu.VMEM((_B, 128), jnp.float32),
        pltpu.VMEM((_CH,), jnp.float32),
        pltpu.VMEM((_CH,), jnp.float32),
        pltpu.SemaphoreType.DMA,
    ],
)
def _sc_fma(x_hbm, idx_hbm, sites_hbm, masks_hbm, out_hbm,
            idx_v, srows, mrows, xbuf, obuf, sem):
    b = lax.axis_index("s") * _NC + lax.axis_index("c")
    pltpu.sync_copy(idx_hbm, idx_v)
    pltpu.async_copy(sites_hbm.at[idx_v], srows, sem).wait()
    pltpu.async_copy(masks_hbm.at[idx_v], mrows, sem).wait()
    sv = jnp.zeros((16,), jnp.float32)
    mv = jnp.zeros((16,), jnp.float32)
    for j in range(_B):
        pred = b == j
        sv = jnp.where(pred, srows[j, pl.ds(0, 16)], sv)
        mv = jnp.where(pred, mrows[j, pl.ds(0, 16)], mv)

    def chunk(c, carry):
        pltpu.sync_copy(x_hbm.at[b, pl.ds(c * _CH, _CH)], xbuf)

        def inner(i, carry2):
            base = i * (16 * _UN)
            for u in range(_UN):
                o = base + u * 16
                obuf[pl.ds(o, 16)] = xbuf[pl.ds(o, 16)] * mv + sv
            return carry2

        lax.fori_loop(0, _CH // (16 * _UN), inner, 0)
        pltpu.sync_copy(obuf, out_hbm.at[b, pl.ds(c * _CH, _CH)])
        return carry

    lax.fori_loop(0, _F // _CH, chunk, 0)


def kernel(inputs, available_injection_sites, masks):
    B, H, W, C = inputs.shape
    n = available_injection_sites.shape[0]
    idx = jax.random.randint(jax.random.key(22), (B,), 0, n).astype(jnp.int32)
    sites16 = jnp.broadcast_to(
        available_injection_sites.reshape(n, 1), (n, 128))
    masks16 = jnp.broadcast_to(masks.reshape(n, 1), (n, 128))
    x = inputs.reshape(B, H * W * C)
    out = _sc_fma(x, idx, sites16, masks16)
    return out.reshape(B, H, W, C)


# speedup vs baseline: 4.5361x; 4.5361x over previous
import jax
import jax.numpy as jnp
from jax.experimental import pallas as pl
from jax.experimental.pallas import tpu as pltpu

_BB = 4


def _fma_body(idx_ref, site_ref, mask_ref, x_ref, o_ref):
    b = pl.program_id(0)
    for j in range(_BB):
        i = idx_ref[b * _BB + j]
        o_ref[j] = x_ref[j] * mask_ref[i] + site_ref[i]


def kernel(inputs, available_injection_sites, masks):
    B, H, W, C = inputs.shape
    n = available_injection_sites.shape[0]
    idx = jax.random.randint(jax.random.key(22), (B,), 0, n).astype(jnp.int32)
    sites = available_injection_sites.reshape(n)
    msk = masks.reshape(n)

    x = inputs.reshape(B, H * W, C)
    out = pl.pallas_call(
        _fma_body,
        grid_spec=pltpu.PrefetchScalarGridSpec(
            num_scalar_prefetch=3,
            grid=(B // _BB,),
            in_specs=[
                pl.BlockSpec((_BB, H * W, C), lambda b, *_: (b, 0, 0)),
            ],
            out_specs=pl.BlockSpec((_BB, H * W, C), lambda b, *_: (b, 0, 0)),
        ),
        out_shape=jax.ShapeDtypeStruct((B, H * W, C), inputs.dtype),
        compiler_params=pltpu.CompilerParams(
            dimension_semantics=("parallel",),
        ),
    )(idx, sites, msk, x)
    return out.reshape(B, H, W, C)
